# baseline (speedup 1.0000x reference)
import functools

import jax
import jax.numpy as jnp
from jax import lax
from jax.experimental import pallas as pl
from jax.experimental.pallas import tpu as pltpu
from jax.experimental.pallas import tpu_sc as plsc

_B, _C = 16384, 1000
_LAMB = max(5.0, 1500.0 / 1.01)
_C1 = _LAMB / (1.0 + _LAMB)
_C2 = 1.0 / (1.0 + _LAMB)

_NC, _NS, _L = 2, 16, 16
_NW = _NC * _NS
_BPW = _B // _NW

_R = 1024
_GRID = _B // _R


def _sc_gather_phi(xphi_flat, target):
    mesh = plsc.VectorSubcoreMesh(core_axis_name="c", subcore_axis_name="s")

    @functools.partial(
        pl.kernel,
        mesh=mesh,
        out_type=jax.ShapeDtypeStruct((_B,), jnp.float32),
        scratch_types=[
            pltpu.VMEM((_BPW,), jnp.int32),
            pltpu.VMEM((_BPW,), jnp.int32),
            pltpu.VMEM((_BPW,), jnp.float32),
            pltpu.SemaphoreType.DMA,
        ],
    )
    def k(xphi_hbm, tgt_hbm, out_hbm, tgt_v, fidx_v, phi_v, sem):
        wid = lax.axis_index("s") * _NC + lax.axis_index("c")
        base = wid * _BPW
        pltpu.sync_copy(tgt_hbm.at[pl.ds(base, _BPW)], tgt_v)

        def body(j, carry):
            t16 = tgt_v[pl.ds(j * _L, _L)]
            rows = base + j * _L + lax.iota(jnp.int32, _L)
            fidx_v[pl.ds(j * _L, _L)] = rows * _C + t16
            return carry

        lax.fori_loop(0, _BPW // _L, body, 0)
        pltpu.async_copy(xphi_hbm.at[fidx_v], phi_v, sem).wait()
        pltpu.sync_copy(phi_v, out_hbm.at[pl.ds(base, _BPW)])

    return k(xphi_flat, target)


def _stats_body(x_ref, t_ref, m_ref, s_ref, c_ref):
    x = x_ref[...]
    t = t_ref[...]
    iota = lax.broadcasted_iota(jnp.int32, (_R, _C), 1)
    mask = iota == t
    cos_t = jnp.sum(jnp.where(mask, x, 0.0), axis=1)
    xe = jnp.where(mask, -jnp.inf, x)
    m = jnp.max(xe, axis=1)
    s = jnp.sum(jnp.exp(xe - m[:, None]), axis=1)
    m_ref[...] = m
    s_ref[...] = s
    c_ref[...] = cos_t


def _combine_body(m_ref, s_ref, c_ref, phi_ref, out_ref):
    m, s, cos_t, phi = m_ref[...], s_ref[...], c_ref[...], phi_ref[...]
    bal = cos_t * _C1 + phi * _C2
    big = jnp.maximum(m, bal)
    ss = s * jnp.exp(m - big) + jnp.exp(bal - big)
    loss = big + jnp.log(ss) - bal
    out_ref[...] = jnp.sum(loss, keepdims=True).reshape(1, 1) * (1.0 / _B)


@jax.jit
def kernel(x_cos, x_phi, target):
    phi_t = _sc_gather_phi(x_phi.reshape(-1), target)
    m, s, cos_t = pl.pallas_call(
        _stats_body,
        grid=(_GRID,),
        in_specs=[
            pl.BlockSpec((_R, _C), lambda i: (i, 0)),
            pl.BlockSpec((_R, 1), lambda i: (i, 0)),
        ],
        out_specs=[
            pl.BlockSpec((_R,), lambda i: (i,)),
            pl.BlockSpec((_R,), lambda i: (i,)),
            pl.BlockSpec((_R,), lambda i: (i,)),
        ],
        out_shape=[
            jax.ShapeDtypeStruct((_B,), jnp.float32),
            jax.ShapeDtypeStruct((_B,), jnp.float32),
            jax.ShapeDtypeStruct((_B,), jnp.float32),
        ],
    )(x_cos, target.reshape(_B, 1))
    loss = pl.pallas_call(
        _combine_body,
        out_shape=jax.ShapeDtypeStruct((1, 1), jnp.float32),
    )(m, s, cos_t, phi_t)
    return loss[0, 0]


# speedup vs baseline: 3.8349x; 3.8349x over previous
import functools

import jax
import jax.numpy as jnp
from jax import lax
from jax.experimental import pallas as pl
from jax.experimental.pallas import tpu as pltpu
from jax.experimental.pallas import tpu_sc as plsc

_B, _C = 16384, 1000
_LAMB = max(5.0, 1500.0 / 1.01)
_C1 = _LAMB / (1.0 + _LAMB)
_C2 = 1.0 / (1.0 + _LAMB)

_NC, _NS, _L = 2, 16, 16
_NW = _NC * _NS
_BPW = _B // _NW
_CHC = 128

_BC = 512
_GRID = _B // _BC


def _sc_gather_phi(xp, target):
    mesh = plsc.VectorSubcoreMesh(core_axis_name="c", subcore_axis_name="s")

    @functools.partial(
        pl.kernel,
        mesh=mesh,
        out_type=jax.ShapeDtypeStruct((_B,), jnp.float32),
        scratch_types=[
            pltpu.VMEM((_BPW,), jnp.int32),
            pltpu.VMEM((_C, _CHC), jnp.float32),
            pltpu.VMEM((_BPW,), jnp.float32),
            pltpu.SemaphoreType.DMA,
        ],
        compiler_params=pltpu.CompilerParams(
            use_tc_tiling_on_sc=True, needs_layout_passes=False
        ),
    )
    def k(xp_hbm, tgt_hbm, out_hbm, tgt_v, chunk_v, phi_v, sem):
        wid = lax.axis_index("s") * _NC + lax.axis_index("c")
        base = wid * _BPW
        pltpu.sync_copy(tgt_hbm.at[pl.ds(base, _BPW)], tgt_v)

        def body(j, carry):
            c0 = j * _CHC
            pltpu.sync_copy(xp_hbm.at[:, pl.ds(base + c0, _CHC)], chunk_v)
            for sub in range(_CHC // _L):
                o = sub * _L
                ccols = o + lax.iota(jnp.int32, _L)
                rows = tgt_v[pl.ds(c0 + o, _L)]
                vals = plsc.load_gather(chunk_v, [rows, ccols])
                phi_v[pl.ds(c0 + o, _L)] = vals
            return carry

        lax.fori_loop(0, _BPW // _CHC, body, 0)
        pltpu.sync_copy(phi_v, out_hbm.at[pl.ds(base, _BPW)])

    return k(xp, target)


def _stats_body(x_ref, t_ref, m_ref, s_ref, c_ref):
    x = x_ref[...]
    t = t_ref[...]
    iota = lax.broadcasted_iota(jnp.int32, (_C, _BC), 0)
    mask = iota == t
    cos_t = jnp.sum(jnp.where(mask, x, 0.0), axis=0, keepdims=True)
    xe = jnp.where(mask, -jnp.inf, x)
    m = jnp.max(xe, axis=0, keepdims=True)
    s = jnp.sum(jnp.exp(xe - m), axis=0, keepdims=True)
    m_ref[...] = m
    s_ref[...] = s
    c_ref[...] = cos_t


def _combine_body(m_ref, s_ref, c_ref, phi_ref, out_ref):
    m, s, cos_t, phi = m_ref[...], s_ref[...], c_ref[...], phi_ref[...]
    bal = cos_t * _C1 + phi * _C2
    big = jnp.maximum(m, bal)
    ss = s * jnp.exp(m - big) + jnp.exp(bal - big)
    loss = big + jnp.log(ss) - bal
    out_ref[...] = jnp.sum(loss, axis=1, keepdims=True) * (1.0 / _B)


@jax.jit
def kernel(x_cos, x_phi, target):
    xc = x_cos.T
    xp = x_phi.T
    t2 = target.reshape(1, _B)
    phi_t = _sc_gather_phi(xp, target)
    m, s, cos_t = pl.pallas_call(
        _stats_body,
        grid=(_GRID,),
        in_specs=[
            pl.BlockSpec((_C, _BC), lambda i: (0, i)),
            pl.BlockSpec((1, _BC), lambda i: (0, i)),
        ],
        out_specs=[
            pl.BlockSpec((1, _BC), lambda i: (0, i)),
            pl.BlockSpec((1, _BC), lambda i: (0, i)),
            pl.BlockSpec((1, _BC), lambda i: (0, i)),
        ],
        out_shape=[
            jax.ShapeDtypeStruct((1, _B), jnp.float32),
            jax.ShapeDtypeStruct((1, _B), jnp.float32),
            jax.ShapeDtypeStruct((1, _B), jnp.float32),
        ],
    )(xc, t2)
    loss = pl.pallas_call(
        _combine_body,
        out_shape=jax.ShapeDtypeStruct((1, 1), jnp.float32),
    )(m, s, cos_t, phi_t.reshape(1, _B))
    return loss[0, 0]


# speedup vs baseline: 3.8496x; 1.0039x over previous
import functools

import jax
import jax.numpy as jnp
from jax import lax
from jax.experimental import pallas as pl
from jax.experimental.pallas import tpu as pltpu
from jax.experimental.pallas import tpu_sc as plsc

_B, _C = 16384, 1000
_LAMB = max(5.0, 1500.0 / 1.01)
_C1 = _LAMB / (1.0 + _LAMB)
_C2 = 1.0 / (1.0 + _LAMB)

_NC, _NS, _L = 2, 16, 16
_NW = _NC * _NS
_BPW = _B // _NW
_CHC = 128

_BC = 512
_GRID = _B // _BC


def _sc_gather_phi(xp, target):
    mesh = plsc.VectorSubcoreMesh(core_axis_name="c", subcore_axis_name="s")

    @functools.partial(
        pl.kernel,
        mesh=mesh,
        out_type=jax.ShapeDtypeStruct((_B,), jnp.float32),
        scratch_types=[
            pltpu.VMEM((_BPW,), jnp.int32),
            pltpu.VMEM((_C, _CHC), jnp.float32),
            pltpu.VMEM((_BPW,), jnp.float32),
            pltpu.SemaphoreType.DMA,
        ],
        compiler_params=pltpu.CompilerParams(
            use_tc_tiling_on_sc=True, needs_layout_passes=False
        ),
    )
    def k(xp_hbm, tgt_hbm, out_hbm, tgt_v, chunk_v, phi_v, sem):
        wid = lax.axis_index("s") * _NC + lax.axis_index("c")
        base = wid * _BPW
        pltpu.sync_copy(tgt_hbm.at[pl.ds(base, _BPW)], tgt_v)

        def body(j, carry):
            c0 = j * _CHC
            pltpu.sync_copy(xp_hbm.at[:, pl.ds(base + c0, _CHC)], chunk_v)
            for sub in range(_CHC // _L):
                o = sub * _L
                ccols = o + lax.iota(jnp.int32, _L)
                rows = tgt_v[pl.ds(c0 + o, _L)]
                vals = plsc.load_gather(chunk_v, [rows, ccols])
                phi_v[pl.ds(c0 + o, _L)] = vals
            return carry

        lax.fori_loop(0, _BPW // _CHC, body, 0)
        pltpu.sync_copy(phi_v, out_hbm.at[pl.ds(base, _BPW)])

    return k(xp, target)


def _stats_body(x_ref, t_ref, m_ref, s_ref, c_ref):
    x = x_ref[...]
    t = t_ref[...]
    iota = lax.broadcasted_iota(jnp.int32, (_C, _BC), 0)
    mask = iota == t
    cos_t = jnp.sum(jnp.where(mask, x, 0.0), axis=0, keepdims=True)
    xe = jnp.where(mask, -jnp.inf, x)
    m = jnp.max(xe, axis=0, keepdims=True)
    s = jnp.sum(jnp.exp(xe - m), axis=0, keepdims=True)
    m_ref[...] = m
    s_ref[...] = s
    c_ref[...] = cos_t


def _combine_body(m_ref, s_ref, c_ref, phi_ref, out_ref):
    m, s, cos_t, phi = m_ref[...], s_ref[...], c_ref[...], phi_ref[...]
    bal = cos_t * _C1 + phi * _C2
    big = jnp.maximum(m, bal)
    ss = s * jnp.exp(m - big) + jnp.exp(bal - big)
    loss = big + jnp.log(ss) - bal
    out_ref[...] = jnp.sum(loss, axis=1, keepdims=True) * (1.0 / _B)


@jax.jit
def kernel(x_cos, x_phi, target):
    xc = x_cos.T
    xp = x_phi.T
    t2 = target.reshape(1, _B)
    phi_t = _sc_gather_phi(xp, target)
    m, s, cos_t = pl.pallas_call(
        _stats_body,
        grid=(_GRID,),
        in_specs=[
            pl.BlockSpec((_C, _BC), lambda i: (0, i)),
            pl.BlockSpec((1, _BC), lambda i: (0, i)),
        ],
        out_specs=[
            pl.BlockSpec((1, _BC), lambda i: (0, i)),
            pl.BlockSpec((1, _BC), lambda i: (0, i)),
            pl.BlockSpec((1, _BC), lambda i: (0, i)),
        ],
        out_shape=[
            jax.ShapeDtypeStruct((1, _B), jnp.float32),
            jax.ShapeDtypeStruct((1, _B), jnp.float32),
            jax.ShapeDtypeStruct((1, _B), jnp.float32),
        ],
        compiler_params=pltpu.CompilerParams(
            dimension_semantics=("parallel",)
        ),
    )(xc, t2)
    loss = pl.pallas_call(
        _combine_body,
        out_shape=jax.ShapeDtypeStruct((1, 1), jnp.float32),
    )(m, s, cos_t, phi_t.reshape(1, _B))
    return loss[0, 0]


# speedup vs baseline: 4.0973x; 1.0643x over previous
import functools

import jax
import jax.numpy as jnp
from jax import lax
from jax.experimental import pallas as pl
from jax.experimental.pallas import tpu as pltpu
from jax.experimental.pallas import tpu_sc as plsc

_B, _C = 16384, 1000
_LAMB = max(5.0, 1500.0 / 1.01)
_C1 = _LAMB / (1.0 + _LAMB)
_C2 = 1.0 / (1.0 + _LAMB)

_NC, _NS, _L = 2, 16, 16
_NW = _NC * _NS
_BPW = _B // _NW
_CHC = 128

_BC = 1024
_GRID = _B // _BC


def _sc_gather_phi(xp, target):
    mesh = plsc.VectorSubcoreMesh(core_axis_name="c", subcore_axis_name="s")

    @functools.partial(
        pl.kernel,
        mesh=mesh,
        out_type=jax.ShapeDtypeStruct((_B,), jnp.float32),
        scratch_types=[
            pltpu.VMEM((_BPW,), jnp.int32),
            pltpu.VMEM((_C, _CHC), jnp.float32),
            pltpu.VMEM((_BPW,), jnp.float32),
            pltpu.SemaphoreType.DMA,
        ],
        compiler_params=pltpu.CompilerParams(
            use_tc_tiling_on_sc=True, needs_layout_passes=False
        ),
    )
    def k(xp_hbm, tgt_hbm, out_hbm, tgt_v, chunk_v, phi_v, sem):
        wid = lax.axis_index("s") * _NC + lax.axis_index("c")
        base = wid * _BPW
        pltpu.sync_copy(tgt_hbm.at[pl.ds(base, _BPW)], tgt_v)

        def body(j, carry):
            c0 = j * _CHC
            pltpu.sync_copy(xp_hbm.at[:, pl.ds(base + c0, _CHC)], chunk_v)
            for sub in range(_CHC // _L):
                o = sub * _L
                ccols = o + lax.iota(jnp.int32, _L)
                rows = tgt_v[pl.ds(c0 + o, _L)]
                vals = plsc.load_gather(chunk_v, [rows, ccols])
                phi_v[pl.ds(c0 + o, _L)] = vals
            return carry

        lax.fori_loop(0, _BPW // _CHC, body, 0)
        pltpu.sync_copy(phi_v, out_hbm.at[pl.ds(base, _BPW)])

    return k(xp, target)


def _stats_body(x_ref, t_ref, m_ref, s_ref, c_ref):
    x = x_ref[...]
    t = t_ref[...]
    iota = lax.broadcasted_iota(jnp.int32, (_C, _BC), 0)
    mask = iota == t
    cos_t = jnp.sum(jnp.where(mask, x, 0.0), axis=0, keepdims=True)
    xe = jnp.where(mask, -jnp.inf, x)
    m = jnp.max(xe, axis=0, keepdims=True)
    s = jnp.sum(jnp.exp(xe - m), axis=0, keepdims=True)
    m_ref[...] = m
    s_ref[...] = s
    c_ref[...] = cos_t


def _combine_body(m_ref, s_ref, c_ref, phi_ref, out_ref):
    m, s, cos_t, phi = m_ref[...], s_ref[...], c_ref[...], phi_ref[...]
    bal = cos_t * _C1 + phi * _C2
    big = jnp.maximum(m, bal)
    ss = s * jnp.exp(m - big) + jnp.exp(bal - big)
    loss = big + jnp.log(ss) - bal
    out_ref[...] = jnp.sum(loss, axis=1, keepdims=True) * (1.0 / _B)


@jax.jit
def kernel(x_cos, x_phi, target):
    xc = x_cos.T
    xp = x_phi.T
    t2 = target.reshape(1, _B)
    phi_t = _sc_gather_phi(xp, target)
    m, s, cos_t = pl.pallas_call(
        _stats_body,
        grid=(_GRID,),
        in_specs=[
            pl.BlockSpec((_C, _BC), lambda i: (0, i)),
            pl.BlockSpec((1, _BC), lambda i: (0, i)),
        ],
        out_specs=[
            pl.BlockSpec((1, _BC), lambda i: (0, i)),
            pl.BlockSpec((1, _BC), lambda i: (0, i)),
            pl.BlockSpec((1, _BC), lambda i: (0, i)),
        ],
        out_shape=[
            jax.ShapeDtypeStruct((1, _B), jnp.float32),
            jax.ShapeDtypeStruct((1, _B), jnp.float32),
            jax.ShapeDtypeStruct((1, _B), jnp.float32),
        ],
        compiler_params=pltpu.CompilerParams(
            dimension_semantics=("parallel",)
        ),
    )(xc, t2)
    loss = pl.pallas_call(
        _combine_body,
        out_shape=jax.ShapeDtypeStruct((1, 1), jnp.float32),
    )(m, s, cos_t, phi_t.reshape(1, _B))
    return loss[0, 0]


# speedup vs baseline: 4.2667x; 1.0413x over previous
import functools

import jax
import jax.numpy as jnp
from jax import lax
from jax.experimental import pallas as pl
from jax.experimental.pallas import tpu as pltpu
from jax.experimental.pallas import tpu_sc as plsc

_B, _C = 16384, 1000
_LAMB = max(5.0, 1500.0 / 1.01)
_C1 = _LAMB / (1.0 + _LAMB)
_C2 = 1.0 / (1.0 + _LAMB)

_NC, _NS, _L = 2, 16, 16
_NW = _NC * _NS
_BPW = _B // _NW
_CHC = 128

_BC = 2048
_GRID = _B // _BC


def _sc_gather_phi(xp, target):
    mesh = plsc.VectorSubcoreMesh(core_axis_name="c", subcore_axis_name="s")

    @functools.partial(
        pl.kernel,
        mesh=mesh,
        out_type=jax.ShapeDtypeStruct((_B,), jnp.float32),
        scratch_types=[
            pltpu.VMEM((_BPW,), jnp.int32),
            pltpu.VMEM((_C, _CHC), jnp.float32),
            pltpu.VMEM((_BPW,), jnp.float32),
            pltpu.SemaphoreType.DMA,
        ],
        compiler_params=pltpu.CompilerParams(
            use_tc_tiling_on_sc=True, needs_layout_passes=False
        ),
    )
    def k(xp_hbm, tgt_hbm, out_hbm, tgt_v, chunk_v, phi_v, sem):
        wid = lax.axis_index("s") * _NC + lax.axis_index("c")
        base = wid * _BPW
        pltpu.sync_copy(tgt_hbm.at[pl.ds(base, _BPW)], tgt_v)

        def body(j, carry):
            c0 = j * _CHC
            pltpu.sync_copy(xp_hbm.at[:, pl.ds(base + c0, _CHC)], chunk_v)
            for sub in range(_CHC // _L):
                o = sub * _L
                ccols = o + lax.iota(jnp.int32, _L)
                rows = tgt_v[pl.ds(c0 + o, _L)]
                vals = plsc.load_gather(chunk_v, [rows, ccols])
                phi_v[pl.ds(c0 + o, _L)] = vals
            return carry

        lax.fori_loop(0, _BPW // _CHC, body, 0)
        pltpu.sync_copy(phi_v, out_hbm.at[pl.ds(base, _BPW)])

    return k(xp, target)


def _stats_body(x_ref, t_ref, m_ref, s_ref, c_ref):
    x = x_ref[...]
    t = t_ref[...]
    iota = lax.broadcasted_iota(jnp.int32, (_C, _BC), 0)
    mask = iota == t
    cos_t = jnp.sum(jnp.where(mask, x, 0.0), axis=0, keepdims=True)
    xe = jnp.where(mask, -jnp.inf, x)
    m = jnp.max(xe, axis=0, keepdims=True)
    s = jnp.sum(jnp.exp(xe - m), axis=0, keepdims=True)
    m_ref[...] = m
    s_ref[...] = s
    c_ref[...] = cos_t


def _combine_body(m_ref, s_ref, c_ref, phi_ref, out_ref):
    m, s, cos_t, phi = m_ref[...], s_ref[...], c_ref[...], phi_ref[...]
    bal = cos_t * _C1 + phi * _C2
    big = jnp.maximum(m, bal)
    ss = s * jnp.exp(m - big) + jnp.exp(bal - big)
    loss = big + jnp.log(ss) - bal
    out_ref[...] = jnp.sum(loss, axis=1, keepdims=True) * (1.0 / _B)


@jax.jit
def kernel(x_cos, x_phi, target):
    xc = x_cos.T
    xp = x_phi.T
    t2 = target.reshape(1, _B)
    phi_t = _sc_gather_phi(xp, target)
    m, s, cos_t = pl.pallas_call(
        _stats_body,
        grid=(_GRID,),
        in_specs=[
            pl.BlockSpec((_C, _BC), lambda i: (0, i)),
            pl.BlockSpec((1, _BC), lambda i: (0, i)),
        ],
        out_specs=[
            pl.BlockSpec((1, _BC), lambda i: (0, i)),
            pl.BlockSpec((1, _BC), lambda i: (0, i)),
            pl.BlockSpec((1, _BC), lambda i: (0, i)),
        ],
        out_shape=[
            jax.ShapeDtypeStruct((1, _B), jnp.float32),
            jax.ShapeDtypeStruct((1, _B), jnp.float32),
            jax.ShapeDtypeStruct((1, _B), jnp.float32),
        ],
        compiler_params=pltpu.CompilerParams(
            dimension_semantics=("parallel",)
        ),
    )(xc, t2)
    loss = pl.pallas_call(
        _combine_body,
        out_shape=jax.ShapeDtypeStruct((1, 1), jnp.float32),
    )(m, s, cos_t, phi_t.reshape(1, _B))
    return loss[0, 0]
